# baseline (speedup 1.0000x reference)
import functools

import jax
import jax.numpy as jnp
from jax import lax
from jax.experimental import pallas as pl
from jax.experimental.pallas import tpu as pltpu
from jax.experimental.pallas import tpu_sc as plsc

_NC = 2
_NS = 16


def _chunk_sizes(total_rows, max_rows):
    sizes = []
    left = total_rows
    first = min(16, left)
    if 0 < first < left:
        sizes.append(first)
        left -= first
    while left > 0:
        c = min(left, max_rows)
        sizes.append(c)
        left -= c
    return sizes


def kernel(x, pos_table):
    batch, seq_len = x.shape[0], x.shape[1]
    d_model = pos_table.shape[1]
    nw = _NC * _NS
    rows_per_w = seq_len // nw
    max_rows = min(rows_per_w, (131064 // (2 * d_model)) // 8 * 8)
    sizes = _chunk_sizes(rows_per_w, max_rows)
    offs = []
    o = 0
    for c in sizes:
        offs.append(o)
        o += c

    mesh = plsc.VectorSubcoreMesh(
        core_axis_name="c",
        subcore_axis_name="s",
        num_cores=_NC,
        num_subcores=_NS,
    )

    @functools.partial(
        pl.kernel,
        out_type=jax.ShapeDtypeStruct((batch, seq_len, d_model), jnp.float32),
        mesh=mesh,
        scratch_types=[
            pltpu.VMEM((max_rows, d_model), jnp.float32),
            pltpu.VMEM((max_rows, d_model), jnp.float32),
            pltpu.SemaphoreType.DMA,
            pltpu.SemaphoreType.DMA,
            pltpu.SemaphoreType.DMA,
            pltpu.SemaphoreType.DMA,
        ],
    )
    def broadcast_rows(table_hbm, out_hbm, buf0, buf1, rs0, rs1, ws0, ws1):
        bufs = (buf0, buf1)
        rsems = (rs0, rs1)
        wsems = (ws0, ws1)
        wid = lax.axis_index("s") * _NC + lax.axis_index("c")
        base = wid * rows_per_w

        reads = [None, None]
        writes = [[], []]
        reads[0] = pltpu.async_copy(table_hbm.at[pl.ds(base + offs[0], sizes[0])],
                                    bufs[0].at[pl.ds(0, sizes[0])], rsems[0])
        for i, c in enumerate(sizes):
            cur = i & 1
            nxt = 1 - cur
            reads[cur].wait()
            if i + 1 < len(sizes):
                for wh in writes[nxt]:
                    wh.wait()
                writes[nxt] = []
                reads[nxt] = pltpu.async_copy(
                    table_hbm.at[pl.ds(base + offs[i + 1], sizes[i + 1])],
                    bufs[nxt].at[pl.ds(0, sizes[i + 1])], rsems[nxt])
            r0 = base + offs[i]
            for b in range(batch):
                writes[cur].append(
                    pltpu.async_copy(bufs[cur].at[pl.ds(0, c)],
                                     out_hbm.at[b, pl.ds(r0, c)], wsems[cur]))
        for lst in writes:
            for wh in lst:
                wh.wait()

    return broadcast_rows(pos_table)


# speedup vs baseline: 1.0371x; 1.0371x over previous
import functools

import jax
import jax.numpy as jnp
from jax import lax
from jax.experimental import pallas as pl
from jax.experimental.pallas import tpu as pltpu
from jax.experimental.pallas import tpu_sc as plsc

_NC = 2
_NS = 16


def _chunk_sizes(total_rows, max_rows):
    sizes = []
    left = total_rows
    while left > 0:
        c = min(left, max_rows)
        sizes.append(c)
        left -= c
    return sizes


def kernel(x, pos_table):
    batch, seq_len = x.shape[0], x.shape[1]
    d_model = pos_table.shape[1]
    nw = _NC * _NS
    rows_per_w = seq_len // nw
    max_rows = min(rows_per_w, (131064 // d_model) // 8 * 8)
    sizes = _chunk_sizes(rows_per_w, max_rows)

    mesh = plsc.VectorSubcoreMesh(
        core_axis_name="c",
        subcore_axis_name="s",
        num_cores=_NC,
        num_subcores=_NS,
    )

    @functools.partial(
        pl.kernel,
        out_type=jax.ShapeDtypeStruct((batch, seq_len, d_model), jnp.float32),
        mesh=mesh,
        scratch_types=[
            pltpu.VMEM((max_rows, d_model), jnp.float32),
            pltpu.SemaphoreType.DMA,
        ],
    )
    def broadcast_rows(table_hbm, out_hbm, buf, rsem):
        wid = lax.axis_index("s") * _NC + lax.axis_index("c")
        base = wid * rows_per_w

        off = 0
        for c in sizes:
            r0 = base + off
            pltpu.async_copy(table_hbm.at[pl.ds(r0, c)],
                             buf.at[pl.ds(0, c)], rsem).wait()
            for b in range(batch):
                pltpu.sync_copy(buf.at[pl.ds(0, c)],
                                out_hbm.at[b, pl.ds(r0, c)])
            off += c
    return broadcast_rows(pos_table)
